# baseline (speedup 1.0000x reference)
import functools

import jax
import jax.numpy as jnp
from jax import lax
from jax.experimental import pallas as pl
from jax.experimental.pallas import tpu as pltpu
from jax.experimental.pallas import tpu_sc as plsc

B = 4096
D = 32
GPR = 4
W = 128
N_ROWS = 1000000
NC = 2
NS = 16
NW = NC * NS
BPW = B // NW


def _sc_gather_body(uidx_hbm, iidx_hbm, uf_hbm, if_hbm, ublk_out, vblk_out,
                    uidx, ublk, iidx, iblk, usem, isem):
    wid = lax.axis_index("s") * NC + lax.axis_index("c")
    base = wid * BPW
    pltpu.sync_copy(uidx_hbm.at[pl.ds(base, BPW)], uidx)
    pltpu.sync_copy(iidx_hbm.at[pl.ds(base, BPW)], iidx)
    cu = pltpu.async_copy(uf_hbm.at[uidx], ublk, usem)
    ci = pltpu.async_copy(if_hbm.at[iidx], iblk, isem)
    cu.wait()
    ci.wait()
    pltpu.sync_copy(ublk, ublk_out.at[pl.ds(base, BPW)])
    pltpu.sync_copy(iblk, vblk_out.at[pl.ds(base, BPW)])


_sc_gather = functools.partial(
    pl.kernel,
    mesh=plsc.VectorSubcoreMesh(core_axis_name="c", subcore_axis_name="s"),
    out_type=[
        jax.ShapeDtypeStruct((B, W), jnp.float32),
        jax.ShapeDtypeStruct((B, W), jnp.float32),
    ],
    scratch_types=[
        pltpu.VMEM((BPW,), jnp.int32),
        pltpu.VMEM((BPW, W), jnp.float32),
        pltpu.VMEM((BPW,), jnp.int32),
        pltpu.VMEM((BPW, W), jnp.float32),
        pltpu.SemaphoreType.DMA,
        pltpu.SemaphoreType.DMA,
    ],
)(_sc_gather_body)


BM = 512
BN = 1024


def _sel_matrix():
    return (lax.broadcasted_iota(jnp.int32, (W, D), 0) % D
            == lax.broadcasted_iota(jnp.int32, (W, D), 1)).astype(jnp.float32)


def _mm_body(ublk_ref, upos_ref, vblk_ref, vpos_ref, o_ref, u_s, v_s):
    i = pl.program_id(0)
    j = pl.program_id(1)
    p = _sel_matrix()

    @pl.when(j == 0)
    def _():
        grp = lax.broadcasted_iota(jnp.int32, (BM, W), 1) // D
        uq = jnp.where(grp == upos_ref[...], ublk_ref[...], 0.0)
        u_s[...] = jnp.dot(uq, p, preferred_element_type=jnp.float32)

    @pl.when(i == 0)
    def _():
        grp = lax.broadcasted_iota(jnp.int32, (BN, W), 1) // D
        vq = jnp.where(grp == vpos_ref[...], vblk_ref[...], 0.0)
        v_s[pl.ds(j * BN, BN), :] = jnp.dot(
            vq, p, preferred_element_type=jnp.float32)

    o_ref[...] = lax.dot_general(
        u_s[...], v_s[pl.ds(j * BN, BN), :],
        (((1,), (1,)), ((), ())),
        preferred_element_type=jnp.float32,
    )


def _tc_matmul(ublk, upos, vblk, vpos):
    return pl.pallas_call(
        _mm_body,
        grid=(B // BM, B // BN),
        in_specs=[
            pl.BlockSpec((BM, W), lambda i, j: (i, 0)),
            pl.BlockSpec((BM, 1), lambda i, j: (i, 0)),
            pl.BlockSpec((BN, W), lambda i, j: (j, 0)),
            pl.BlockSpec((BN, 1), lambda i, j: (j, 0)),
        ],
        out_specs=pl.BlockSpec((BM, BN), lambda i, j: (i, j)),
        out_shape=jax.ShapeDtypeStruct((B, B), jnp.float32),
        scratch_shapes=[
            pltpu.VMEM((BM, D), jnp.float32),
            pltpu.VMEM((B, D), jnp.float32),
        ],
    )(ublk, upos, vblk, vpos)


def kernel(users, items, user_factors, item_factors):
    users = users.astype(jnp.int32)
    items = items.astype(jnp.int32)
    uf2 = user_factors.reshape(N_ROWS // GPR, W)
    if2 = item_factors.reshape(N_ROWS // GPR, W)
    ublk, vblk = _sc_gather(users // GPR, items // GPR, uf2, if2)
    return _tc_matmul(ublk, (users % GPR).reshape(B, 1),
                      vblk, (items % GPR).reshape(B, 1))


# speedup vs baseline: 1.4916x; 1.4916x over previous
import functools

import jax
import jax.numpy as jnp
from jax import lax
from jax.experimental import pallas as pl
from jax.experimental.pallas import tpu as pltpu
from jax.experimental.pallas import tpu_sc as plsc

B = 4096
D = 32
NC = 2
NS = 16
NW = NC * NS
BPW = B // NW
L = 16


def _gather_rows(idx_ref, table_hbm, rows, sem):
    copies = []
    for c in range(BPW // L):
        chunk = idx_ref[pl.ds(c * L, L)]
        for l in range(L):
            r = jnp.sum(jnp.where(lax.iota(jnp.int32, L) == l, chunk, 0))
            p = c * L + l
            copies.append(pltpu.async_copy(
                table_hbm.at[pl.ds(r, 1)], rows.at[pl.ds(p, 1)], sem))
    for cp in copies:
        cp.wait()


def _sc_gather_body(users_hbm, items_hbm, uf_hbm, if_hbm, u_out, v_out,
                    uidx, urows, iidx, irows, usem, isem):
    wid = lax.axis_index("s") * NC + lax.axis_index("c")
    base = wid * BPW
    pltpu.sync_copy(users_hbm.at[pl.ds(base, BPW)], uidx)
    pltpu.sync_copy(items_hbm.at[pl.ds(base, BPW)], iidx)
    _gather_rows(uidx, uf_hbm, urows, usem)
    _gather_rows(iidx, if_hbm, irows, isem)
    pltpu.sync_copy(urows, u_out.at[pl.ds(base, BPW)])
    pltpu.sync_copy(irows, v_out.at[pl.ds(base, BPW)])


_sc_gather = functools.partial(
    pl.kernel,
    mesh=plsc.VectorSubcoreMesh(core_axis_name="c", subcore_axis_name="s"),
    out_type=[
        jax.ShapeDtypeStruct((B, D), jnp.float32),
        jax.ShapeDtypeStruct((B, D), jnp.float32),
    ],
    scratch_types=[
        pltpu.VMEM((BPW,), jnp.int32),
        pltpu.VMEM((BPW, D), jnp.float32),
        pltpu.VMEM((BPW,), jnp.int32),
        pltpu.VMEM((BPW, D), jnp.float32),
        pltpu.SemaphoreType.DMA,
        pltpu.SemaphoreType.DMA,
    ],
    compiler_params=pltpu.CompilerParams(needs_layout_passes=False),
)(_sc_gather_body)


def _mm_body(u_ref, v_ref, o_ref):
    o_ref[...] = lax.dot_general(
        u_ref[...], v_ref[...],
        (((1,), (1,)), ((), ())),
        preferred_element_type=jnp.float32,
    )


BM = 512
BN = 1024


def _tc_matmul(u, v):
    return pl.pallas_call(
        _mm_body,
        grid=(B // BM, B // BN),
        in_specs=[
            pl.BlockSpec((BM, D), lambda i, j: (i, 0)),
            pl.BlockSpec((BN, D), lambda i, j: (j, 0)),
        ],
        out_specs=pl.BlockSpec((BM, BN), lambda i, j: (i, j)),
        out_shape=jax.ShapeDtypeStruct((B, B), jnp.float32),
    )(u, v)


def kernel(users, items, user_factors, item_factors):
    u, v = _sc_gather(users.astype(jnp.int32), items.astype(jnp.int32),
                      user_factors, item_factors)
    return _tc_matmul(u, v)
